# baseline (speedup 1.0000x reference)
import functools

import jax
import jax.numpy as jnp
from jax import lax
from jax.experimental import pallas as pl
from jax.experimental.pallas import tpu as pltpu
from jax.experimental.pallas import tpu_sc as plsc

_HIDDEN = 128
_N_NODES = 10000
_N_EDGES = 320000

_NC = 2
_NS = 16
_L = 16
_NW = _NC * _NS
_EPW = _N_EDGES // _NW
_UNROLL = 5


def _proj_body(xf_ref, xt_ref, w_ref, b_ref, pf_ref, pt_ref):
    wf = w_ref[:, :_HIDDEN]
    wt = w_ref[:, _HIDDEN:]
    dn = (((1,), (1,)), ((), ()))
    pf_ref[...] = (
        lax.dot_general(wf, xf_ref[...], dn, preferred_element_type=jnp.float32)
        + b_ref[0, 0]
    )
    pt_ref[...] = lax.dot_general(
        wt, xt_ref[...], dn, preferred_element_type=jnp.float32
    )


_project = pl.pallas_call(
    _proj_body,
    out_shape=[
        jax.ShapeDtypeStruct((1, _N_NODES), jnp.float32),
        jax.ShapeDtypeStruct((1, _N_NODES), jnp.float32),
    ],
    in_specs=[
        pl.BlockSpec(memory_space=pltpu.VMEM),
        pl.BlockSpec(memory_space=pltpu.VMEM),
        pl.BlockSpec(memory_space=pltpu.VMEM),
        pl.BlockSpec(memory_space=pltpu.SMEM),
    ],
    out_specs=[
        pl.BlockSpec(memory_space=pltpu.VMEM),
        pl.BlockSpec(memory_space=pltpu.VMEM),
    ],
)

_mesh = plsc.VectorSubcoreMesh(
    core_axis_name="c", subcore_axis_name="s", num_cores=_NC, num_subcores=_NS
)


@functools.partial(
    pl.kernel,
    mesh=_mesh,
    compiler_params=pltpu.CompilerParams(
        needs_layout_passes=False,
        disable_bounds_checks=True,
        disable_semaphore_checks=True,
        skip_device_barrier=True,
    ),
    out_type=jax.ShapeDtypeStruct((_N_EDGES,), jnp.float32),
    scratch_types=[
        pltpu.VMEM((_N_NODES,), jnp.float32),
        pltpu.VMEM((_N_NODES,), jnp.float32),
        pltpu.VMEM((_EPW,), jnp.int32),
        pltpu.VMEM((_EPW,), jnp.int32),
        pltpu.VMEM((_EPW,), jnp.float32),
    ],
)
def _edge_gather(pf_hbm, pt_hbm, i0_hbm, i1_hbm, out_hbm,
                 pf_v, pt_v, i0_v, i1_v, out_v):
    wid = lax.axis_index("s") * _NC + lax.axis_index("c")
    base = wid * _EPW
    pltpu.sync_copy(pf_hbm, pf_v)
    pltpu.sync_copy(pt_hbm, pt_v)
    pltpu.sync_copy(i0_hbm.at[pl.ds(base, _EPW)], i0_v)
    pltpu.sync_copy(i1_hbm.at[pl.ds(base, _EPW)], i1_v)

    def body(i, carry):
        base_u = i * (_L * _UNROLL)
        for u in range(_UNROLL):
            sl = pl.ds(base_u + u * _L, _L)
            a = plsc.load_gather(pf_v, [i0_v[sl]])
            c = plsc.load_gather(pt_v, [i1_v[sl]])
            out_v[sl] = a + c
        return carry

    lax.fori_loop(0, _EPW // (_L * _UNROLL), body, 0)
    pltpu.sync_copy(out_v, out_hbm.at[pl.ds(base, _EPW)])


def kernel(x_from, x_to, edge_label_index, W, b):
    pf, pt = _project(x_from, x_to, W, b.reshape(1, 1))
    idx = edge_label_index.astype(jnp.int32)
    return _edge_gather(
        pf.reshape(_N_NODES), pt.reshape(_N_NODES), idx[0], idx[1]
    )


# speedup vs baseline: 1.0488x; 1.0488x over previous
import functools

import jax
import jax.numpy as jnp
from jax import lax
from jax.experimental import pallas as pl
from jax.experimental.pallas import tpu as pltpu
from jax.experimental.pallas import tpu_sc as plsc

_HIDDEN = 128
_N_NODES = 10000
_N_EDGES = 320000

_NC = 2
_NS = 16
_L = 16
_NW = _NC * _NS
_EPW = _N_EDGES // _NW
_UNROLL = 5


def _proj_body(xf_ref, xt_ref, w_ref, b_ref, pf_ref, pt_ref):
    wf = w_ref[:, :_HIDDEN]
    wt = w_ref[:, _HIDDEN:]
    dn = (((1,), (1,)), ((), ()))
    pf_ref[...] = (
        lax.dot_general(wf, xf_ref[...], dn, preferred_element_type=jnp.float32)
        + b_ref[0, 0]
    )
    pt_ref[...] = lax.dot_general(
        wt, xt_ref[...], dn, preferred_element_type=jnp.float32
    )


_project = pl.pallas_call(
    _proj_body,
    out_shape=[
        jax.ShapeDtypeStruct((1, _N_NODES), jnp.float32),
        jax.ShapeDtypeStruct((1, _N_NODES), jnp.float32),
    ],
    in_specs=[
        pl.BlockSpec(memory_space=pltpu.VMEM),
        pl.BlockSpec(memory_space=pltpu.VMEM),
        pl.BlockSpec(memory_space=pltpu.VMEM),
        pl.BlockSpec(memory_space=pltpu.SMEM),
    ],
    out_specs=[
        pl.BlockSpec(memory_space=pltpu.VMEM),
        pl.BlockSpec(memory_space=pltpu.VMEM),
    ],
)

_mesh = plsc.VectorSubcoreMesh(
    core_axis_name="c", subcore_axis_name="s", num_cores=_NC, num_subcores=_NS
)


@functools.partial(
    pl.kernel,
    mesh=_mesh,
    compiler_params=pltpu.CompilerParams(needs_layout_passes=False),
    out_type=jax.ShapeDtypeStruct((_N_EDGES,), jnp.float32),
    scratch_types=[
        pltpu.VMEM((_N_NODES,), jnp.float32),
        pltpu.VMEM((_N_NODES,), jnp.float32),
        pltpu.VMEM((_EPW,), jnp.int32),
        pltpu.VMEM((_EPW,), jnp.int32),
        pltpu.VMEM((_EPW,), jnp.float32),
    ],
)
def _edge_gather(pf_hbm, pt_hbm, i0_hbm, i1_hbm, out_hbm,
                 pf_v, pt_v, i0_v, i1_v, out_v):
    wid = lax.axis_index("s") * _NC + lax.axis_index("c")
    base = wid * _EPW
    pltpu.sync_copy(pf_hbm, pf_v)
    pltpu.sync_copy(pt_hbm, pt_v)
    pltpu.sync_copy(i0_hbm.at[pl.ds(base, _EPW)], i0_v)
    pltpu.sync_copy(i1_hbm.at[pl.ds(base, _EPW)], i1_v)

    @plsc.parallel_loop(0, _EPW, _L * _UNROLL)
    def _gather_loop(i):
        for u in range(_UNROLL):
            sl = pl.ds(i + u * _L, _L)
            a = plsc.load_gather(pf_v, [i0_v[sl]])
            c = plsc.load_gather(pt_v, [i1_v[sl]])
            out_v[sl] = a + c
    pltpu.sync_copy(out_v, out_hbm.at[pl.ds(base, _EPW)])


def kernel(x_from, x_to, edge_label_index, W, b):
    pf, pt = _project(x_from, x_to, W, b.reshape(1, 1))
    idx = edge_label_index.astype(jnp.int32)
    return _edge_gather(
        pf.reshape(_N_NODES), pt.reshape(_N_NODES), idx[0], idx[1]
    )


# speedup vs baseline: 1.0717x; 1.0219x over previous
import functools

import jax
import jax.numpy as jnp
from jax import lax
from jax.experimental import pallas as pl
from jax.experimental.pallas import tpu as pltpu
from jax.experimental.pallas import tpu_sc as plsc

_HIDDEN = 128
_N_NODES = 10000
_N_EDGES = 320000

_NC = 2
_NS = 16
_L = 16
_NW = _NC * _NS
_EPW = _N_EDGES // _NW
_UNROLL = 5


def _proj_body(xf_ref, xt_ref, w_ref, b_ref, p_ref):
    wf = w_ref[:, :_HIDDEN]
    wt = w_ref[:, _HIDDEN:]
    dn = (((1,), (1,)), ((), ()))
    p_ref[:, :_N_NODES] = (
        lax.dot_general(wf, xf_ref[...], dn, preferred_element_type=jnp.float32)
        + b_ref[0, 0]
    )
    p_ref[:, _N_NODES:] = lax.dot_general(
        wt, xt_ref[...], dn, preferred_element_type=jnp.float32
    )


_project = pl.pallas_call(
    _proj_body,
    out_shape=jax.ShapeDtypeStruct((1, 2 * _N_NODES), jnp.float32),
    in_specs=[
        pl.BlockSpec(memory_space=pltpu.VMEM),
        pl.BlockSpec(memory_space=pltpu.VMEM),
        pl.BlockSpec(memory_space=pltpu.VMEM),
        pl.BlockSpec(memory_space=pltpu.SMEM),
    ],
    out_specs=pl.BlockSpec(memory_space=pltpu.VMEM),
)

_mesh = plsc.VectorSubcoreMesh(
    core_axis_name="c", subcore_axis_name="s", num_cores=_NC, num_subcores=_NS
)


@functools.partial(
    pl.kernel,
    mesh=_mesh,
    compiler_params=pltpu.CompilerParams(needs_layout_passes=False),
    out_type=jax.ShapeDtypeStruct((_N_EDGES,), jnp.float32),
    scratch_types=[
        pltpu.VMEM((2 * _N_NODES,), jnp.float32),
        pltpu.VMEM((_EPW,), jnp.int32),
        pltpu.VMEM((_EPW,), jnp.int32),
        pltpu.VMEM((_EPW,), jnp.float32),
        pltpu.SemaphoreType.DMA,
    ],
)
def _edge_gather(p_hbm, i0_hbm, i1_hbm, out_hbm,
                 p_v, i0_v, i1_v, out_v, sem):
    wid = lax.axis_index("s") * _NC + lax.axis_index("c")
    base = wid * _EPW
    cp_p = pltpu.async_copy(p_hbm, p_v, sem)
    cp_i0 = pltpu.async_copy(i0_hbm.at[pl.ds(base, _EPW)], i0_v, sem)
    cp_i1 = pltpu.async_copy(i1_hbm.at[pl.ds(base, _EPW)], i1_v, sem)
    cp_p.wait()
    cp_i0.wait()
    cp_i1.wait()

    @plsc.parallel_loop(0, _EPW, _L * _UNROLL)
    def _gather_loop(i):
        for u in range(_UNROLL):
            sl = pl.ds(i + u * _L, _L)
            a = plsc.load_gather(p_v, [i0_v[sl]])
            c = plsc.load_gather(p_v, [i1_v[sl] + _N_NODES])
            out_v[sl] = a + c
    pltpu.sync_copy(out_v, out_hbm.at[pl.ds(base, _EPW)])


def kernel(x_from, x_to, edge_label_index, W, b):
    p = _project(x_from, x_to, W, b.reshape(1, 1))
    idx = edge_label_index.astype(jnp.int32)
    return _edge_gather(p.reshape(2 * _N_NODES), idx[0], idx[1])
